# baseline (speedup 1.0000x reference)
import functools

import jax
import jax.numpy as jnp
from jax import lax
from jax.experimental import pallas as pl
from jax.experimental.pallas import tpu as pltpu
from jax.experimental.pallas import tpu_sc as plsc

NC = 2
NS = 16
K = 128
DW = 8
PAD = 8


def _mesh():
    return plsc.VectorSubcoreMesh(core_axis_name="c", subcore_axis_name="s")


_SC_PARAMS = pltpu.CompilerParams(use_tc_tiling_on_sc=False)


def _per_tile_rows(sid, n, body_fn):
    b = (n // NS) // 8 * 8
    last = n - b * (NS - 1)

    @pl.when(sid < NS - 1)
    def _():
        body_fn(pl.multiple_of(sid * b, 8), b)

    @pl.when(sid == NS - 1)
    def _():
        body_fn((NS - 1) * b, last)


def _deg_partials(dst3, zeros_dw, ones_dw, n):
    ch = dst3.shape[1]

    @functools.partial(
        pl.kernel,
        out_type=jax.ShapeDtypeStruct((NC, n, DW), jnp.float32),
        mesh=_mesh(),
        compiler_params=_SC_PARAMS,
        scratch_types=[
            pltpu.VMEM_SHARED((n + PAD, DW), jnp.float32),
            pltpu.VMEM((ch, K), jnp.int32),
            pltpu.VMEM((K, DW), jnp.float32),
        ],
    )
    def deg_k(dst_hbm, zero_hbm, ones_hbm, out_hbm, acc, didx, ones_v):
        cid = lax.axis_index("c")
        sid = lax.axis_index("s")
        wid = sid * NC + cid
        pltpu.sync_copy(dst_hbm.at[wid], didx)
        pltpu.sync_copy(ones_hbm, ones_v)
        _per_tile_rows(sid, n, lambda base, sz: pltpu.sync_copy(
            zero_hbm.at[pl.ds(base, sz), :], acc.at[pl.ds(base, sz), :]))
        plsc.subcore_barrier()

        @pl.loop(0, ch)
        def _(ci):
            pltpu.sync_copy(ones_v, acc.at[didx.at[ci]], add=True)

        plsc.subcore_barrier()
        _per_tile_rows(sid, n, lambda base, sz: pltpu.sync_copy(
            acc.at[pl.ds(base, sz), :], out_hbm.at[cid, pl.ds(base, sz), :]))

    return deg_k(dst3, zeros_dw, ones_dw)


def _agg_partials(src3, dst3, hs, zeros_nd, n, d):
    ch = src3.shape[1]
    cb = 16
    nb = ch // cb
    assert ch % cb == 0 and cb % 2 == 0

    @functools.partial(
        pl.kernel,
        out_type=jax.ShapeDtypeStruct((NC, n, d), jnp.float32),
        mesh=_mesh(),
        compiler_params=_SC_PARAMS,
        scratch_types=[
            pltpu.VMEM_SHARED((n + PAD, d), jnp.float32),
            pltpu.VMEM((cb, K), jnp.int32),
            pltpu.VMEM((cb, K), jnp.int32),
            pltpu.VMEM((K, d), jnp.float32),
            pltpu.VMEM((K, d), jnp.float32),
            pltpu.SemaphoreType.DMA,
            pltpu.SemaphoreType.DMA,
        ],
    )
    def agg_k(src_hbm, dst_hbm, hs_hbm, zero_hbm, out_hbm,
              acc, sidx, didx, rows0, rows1, sem0, sem1):
        cid = lax.axis_index("c")
        sid = lax.axis_index("s")
        wid = sid * NC + cid
        @pl.when(cid == 0)
        def _():
            _per_tile_rows(sid, n, lambda base, sz: pltpu.sync_copy(
                hs_hbm.at[pl.ds(base, sz), :], acc.at[pl.ds(base, sz), :]))

        @pl.when(cid != 0)
        def _():
            _per_tile_rows(sid, n, lambda base, sz: pltpu.sync_copy(
                zero_hbm.at[pl.ds(base, sz), :], acc.at[pl.ds(base, sz), :]))

        plsc.subcore_barrier()

        @pl.loop(0, nb)
        def _(b):
            boff = pl.multiple_of(b * cb, 8)
            pltpu.sync_copy(src_hbm.at[wid, pl.ds(boff, cb)], sidx)
            pltpu.sync_copy(dst_hbm.at[wid, pl.ds(boff, cb)], didx)
            pltpu.async_copy(hs_hbm.at[sidx.at[0]], rows0, sem0)

            @pl.loop(0, cb, step=2)
            def _(ci):
                pltpu.async_copy(hs_hbm.at[sidx.at[ci + 1]], rows1, sem1)
                pltpu.make_async_copy(hs_hbm.at[sidx.at[ci]], rows0, sem0).wait()
                pltpu.sync_copy(rows0, acc.at[didx.at[ci]], add=True)

                @pl.when(ci + 2 < cb)
                def _():
                    pltpu.async_copy(hs_hbm.at[sidx.at[ci + 2]], rows0, sem0)

                pltpu.make_async_copy(hs_hbm.at[sidx.at[ci + 1]], rows1, sem1).wait()
                pltpu.sync_copy(rows1, acc.at[didx.at[ci + 1]], add=True)

        plsc.subcore_barrier()
        _per_tile_rows(sid, n, lambda base, sz: pltpu.sync_copy(
            acc.at[pl.ds(base, sz), :], out_hbm.at[cid, pl.ds(base, sz), :]))

    return agg_k(src3, dst3, hs, zeros_nd)


def _dinv_col(deg_ref):
    deg = deg_ref[0, :, 0:1] + deg_ref[1, :, 0:1] + 1.0
    return lax.rsqrt(deg)


def _tc_first(deg_p, x, w1):
    n = x.shape[0]
    dh = w1.shape[1]

    def body(deg_ref, x_ref, w_ref, o_ref):
        dinv = _dinv_col(deg_ref)
        h = jnp.dot(x_ref[...], w_ref[...], preferred_element_type=jnp.float32)
        o_ref[...] = h * dinv

    return pl.pallas_call(
        body, out_shape=jax.ShapeDtypeStruct((n, dh), jnp.float32)
    )(deg_p, x, w1)


def _tc_mid(deg_p, p1, b1, w2):
    n = p1.shape[1]
    do = w2.shape[1]

    def body(deg_ref, p_ref, b_ref, w_ref, o_ref):
        dinv = _dinv_col(deg_ref)
        s = p_ref[0] + p_ref[1]
        h = jnp.maximum(s * dinv + b_ref[...], 0.0)
        h2 = jnp.dot(h, w_ref[...], preferred_element_type=jnp.float32)
        o_ref[...] = h2 * dinv

    return pl.pallas_call(
        body, out_shape=jax.ShapeDtypeStruct((n, do), jnp.float32)
    )(deg_p, p1, b1, w2)


def _tc_last(deg_p, p2, b2):
    n = p2.shape[1]
    do = p2.shape[2]

    def body(deg_ref, p_ref, b_ref, o_ref):
        dinv = _dinv_col(deg_ref)
        o_ref[...] = (p_ref[0] + p_ref[1]) * dinv + b_ref[...]

    return pl.pallas_call(
        body, out_shape=jax.ShapeDtypeStruct((n, do), jnp.float32)
    )(deg_p, p2, b2)


def kernel(x, edge_index, W1, b1, W2, b2):
    n = x.shape[0]
    dh = W1.shape[1]
    do = W2.shape[1]
    e = edge_index.shape[1]

    nw = NC * NS
    ch = -(-e // (nw * K))
    ch = -(-ch // 16) * 16
    pad = nw * ch * K - e
    src = jnp.concatenate([edge_index[0], jnp.zeros((pad,), edge_index.dtype)])
    dst = jnp.concatenate([edge_index[1], jnp.full((pad,), n, edge_index.dtype)])
    src3 = src.reshape(nw, ch, K)
    dst3 = dst.reshape(nw, ch, K)

    zeros_dw = jnp.zeros((n, DW), jnp.float32)
    ones_dw = jnp.ones((K, DW), jnp.float32)
    zeros_h = jnp.zeros((n, dh), jnp.float32)
    zeros_o = jnp.zeros((n, do), jnp.float32)

    deg_p = _deg_partials(dst3, zeros_dw, ones_dw, n)
    h1s = _tc_first(deg_p, x, W1)
    p1 = _agg_partials(src3, dst3, h1s, zeros_h, n, dh)
    h2s = _tc_mid(deg_p, p1, b1, W2)
    p2 = _agg_partials(src3, dst3, h2s, zeros_o, n, do)
    return _tc_last(deg_p, p2, b2)


# speedup vs baseline: 1.0970x; 1.0970x over previous
import functools

import jax
import jax.numpy as jnp
from jax import lax
from jax.experimental import pallas as pl
from jax.experimental.pallas import tpu as pltpu
from jax.experimental.pallas import tpu_sc as plsc

NC = 2
NS = 16
K = 128
DW = 8
DUMMY = 1024


def _mesh():
    return plsc.VectorSubcoreMesh(core_axis_name="c", subcore_axis_name="s")


_SC_PARAMS = pltpu.CompilerParams(use_tc_tiling_on_sc=False)


def _per_tile_rows(sid, n, body_fn):
    b = (n // NS) // 8 * 8
    last = n - b * (NS - 1)

    @pl.when(sid < NS - 1)
    def _():
        body_fn(pl.multiple_of(sid * b, 8), b)

    @pl.when(sid == NS - 1)
    def _():
        body_fn((NS - 1) * b, last)


def _deg_partials(dst3, zeros_dw, ones_dw, n):
    ch = dst3.shape[1]

    @functools.partial(
        pl.kernel,
        out_type=jax.ShapeDtypeStruct((NC, n, DW), jnp.float32),
        mesh=_mesh(),
        compiler_params=_SC_PARAMS,
        scratch_types=[
            pltpu.VMEM_SHARED((n + DUMMY, DW), jnp.float32),
            pltpu.VMEM((ch, K), jnp.int32),
            pltpu.VMEM((K, DW), jnp.float32),
        ],
    )
    def deg_k(dst_hbm, zero_hbm, ones_hbm, out_hbm, acc, didx, ones_v):
        cid = lax.axis_index("c")
        sid = lax.axis_index("s")
        wid = sid * NC + cid
        pltpu.sync_copy(dst_hbm.at[wid], didx)
        pltpu.sync_copy(ones_hbm, ones_v)
        _per_tile_rows(sid, n, lambda base, sz: pltpu.sync_copy(
            zero_hbm.at[pl.ds(base, sz), :], acc.at[pl.ds(base, sz), :]))
        plsc.subcore_barrier()

        @pl.loop(0, ch)
        def _(ci):
            pltpu.sync_copy(ones_v, acc.at[didx.at[ci]], add=True)

        plsc.subcore_barrier()
        _per_tile_rows(sid, n, lambda base, sz: pltpu.sync_copy(
            acc.at[pl.ds(base, sz), :], out_hbm.at[cid, pl.ds(base, sz), :]))

    return deg_k(dst3, zeros_dw, ones_dw)


def _agg_partials(src3, dst3, hs, zeros_nd, n, d):
    ch = src3.shape[1]
    cb = 16
    nb = ch // cb
    assert ch % cb == 0 and cb % 2 == 0

    @functools.partial(
        pl.kernel,
        out_type=jax.ShapeDtypeStruct((NC, n, d), jnp.float32),
        mesh=_mesh(),
        compiler_params=_SC_PARAMS,
        scratch_types=[
            pltpu.VMEM_SHARED((n + DUMMY, d), jnp.float32),
            pltpu.VMEM((cb, K), jnp.int32),
            pltpu.VMEM((cb, K), jnp.int32),
            pltpu.VMEM((K, d), jnp.float32),
            pltpu.VMEM((K, d), jnp.float32),
            pltpu.SemaphoreType.DMA,
            pltpu.SemaphoreType.DMA,
        ],
    )
    def agg_k(src_hbm, dst_hbm, hs_hbm, zero_hbm, out_hbm,
              acc, sidx, didx, rows0, rows1, sem0, sem1):
        cid = lax.axis_index("c")
        sid = lax.axis_index("s")
        wid = sid * NC + cid
        _per_tile_rows(sid, n, lambda base, sz: pltpu.sync_copy(
            zero_hbm.at[pl.ds(base, sz), :], acc.at[pl.ds(base, sz), :]))

        plsc.subcore_barrier()

        @pl.loop(0, nb)
        def _(b):
            boff = pl.multiple_of(b * cb, 8)
            pltpu.sync_copy(src_hbm.at[wid, pl.ds(boff, cb)], sidx)
            pltpu.sync_copy(dst_hbm.at[wid, pl.ds(boff, cb)], didx)
            pltpu.async_copy(hs_hbm.at[sidx.at[0]], rows0, sem0)

            @pl.loop(0, cb, step=2)
            def _(ci):
                pltpu.async_copy(hs_hbm.at[sidx.at[ci + 1]], rows1, sem1)
                pltpu.make_async_copy(hs_hbm.at[sidx.at[ci]], rows0, sem0).wait()
                pltpu.sync_copy(rows0, acc.at[didx.at[ci]], add=True)

                @pl.when(ci + 2 < cb)
                def _():
                    pltpu.async_copy(hs_hbm.at[sidx.at[ci + 2]], rows0, sem0)

                pltpu.make_async_copy(hs_hbm.at[sidx.at[ci + 1]], rows1, sem1).wait()
                pltpu.sync_copy(rows1, acc.at[didx.at[ci + 1]], add=True)

        plsc.subcore_barrier()
        _per_tile_rows(sid, n, lambda base, sz: pltpu.sync_copy(
            acc.at[pl.ds(base, sz), :], out_hbm.at[cid, pl.ds(base, sz), :]))

    return agg_k(src3, dst3, hs, zeros_nd)


def _dinv_col(deg_ref):
    deg = deg_ref[0, :, 0:1] + deg_ref[1, :, 0:1] + 1.0
    return lax.rsqrt(deg)


def _tc_first(deg_p, x, w1):
    n = x.shape[0]
    dh = w1.shape[1]

    def body(deg_ref, x_ref, w_ref, o_ref):
        dinv = _dinv_col(deg_ref)
        h = jnp.dot(x_ref[...], w_ref[...], preferred_element_type=jnp.float32)
        o_ref[...] = h * dinv

    return pl.pallas_call(
        body, out_shape=jax.ShapeDtypeStruct((n, dh), jnp.float32)
    )(deg_p, x, w1)


def _tc_mid(deg_p, p1, h1s, b1, w2):
    n = p1.shape[1]
    do = w2.shape[1]

    def body(deg_ref, p_ref, hs_ref, b_ref, w_ref, o_ref):
        dinv = _dinv_col(deg_ref)
        s = p_ref[0] + p_ref[1] + hs_ref[...]
        h = jnp.maximum(s * dinv + b_ref[...], 0.0)
        h2 = jnp.dot(h, w_ref[...], preferred_element_type=jnp.float32)
        o_ref[...] = h2 * dinv

    return pl.pallas_call(
        body, out_shape=jax.ShapeDtypeStruct((n, do), jnp.float32)
    )(deg_p, p1, h1s, b1, w2)


def _tc_last(deg_p, p2, h2s, b2):
    n = p2.shape[1]
    do = p2.shape[2]

    def body(deg_ref, p_ref, hs_ref, b_ref, o_ref):
        dinv = _dinv_col(deg_ref)
        o_ref[...] = (p_ref[0] + p_ref[1] + hs_ref[...]) * dinv + b_ref[...]

    return pl.pallas_call(
        body, out_shape=jax.ShapeDtypeStruct((n, do), jnp.float32)
    )(deg_p, p2, h2s, b2)


def kernel(x, edge_index, W1, b1, W2, b2):
    n = x.shape[0]
    dh = W1.shape[1]
    do = W2.shape[1]
    e = edge_index.shape[1]

    nw = NC * NS
    ch = -(-e // (nw * K))
    ch = -(-ch // 16) * 16
    pad = nw * ch * K - e
    src = jnp.concatenate([edge_index[0], jnp.zeros((pad,), edge_index.dtype)])
    pad_dst = n + (jnp.arange(pad, dtype=edge_index.dtype) % DUMMY)
    dst = jnp.concatenate([edge_index[1], pad_dst])
    src3 = src.reshape(nw, ch, K)
    dst3 = dst.reshape(nw, ch, K)

    zeros_dw = jnp.zeros((n, DW), jnp.float32)
    ones_dw = jnp.ones((K, DW), jnp.float32)
    zeros_h = jnp.zeros((n, dh), jnp.float32)
    zeros_o = jnp.zeros((n, do), jnp.float32)

    deg_p = _deg_partials(dst3, zeros_dw, ones_dw, n)
    h1s = _tc_first(deg_p, x, W1)
    p1 = _agg_partials(src3, dst3, h1s, zeros_h, n, dh)
    h2s = _tc_mid(deg_p, p1, h1s, b1, W2)
    p2 = _agg_partials(src3, dst3, h2s, zeros_o, n, do)
    return _tc_last(deg_p, p2, h2s, b2)


# speedup vs baseline: 2.5731x; 2.3457x over previous
import functools

import jax
import jax.numpy as jnp
from jax import lax
from jax.experimental import pallas as pl
from jax.experimental.pallas import tpu as pltpu
from jax.experimental.pallas import tpu_sc as plsc

NC = 2
NS = 16
K = 128
DW = 8
DUMMY = 1024


def _mesh():
    return plsc.VectorSubcoreMesh(core_axis_name="c", subcore_axis_name="s")


_SC_PARAMS = pltpu.CompilerParams(use_tc_tiling_on_sc=False)


def _per_tile_rows(sid, n, body_fn):
    b = (n // NS) // 8 * 8
    last = n - b * (NS - 1)

    @pl.when(sid < NS - 1)
    def _():
        body_fn(pl.multiple_of(sid * b, 8), b)

    @pl.when(sid == NS - 1)
    def _():
        body_fn((NS - 1) * b, last)


def _deg_partials(dst3, zeros_dw, ones_dw, n):
    ch = dst3.shape[1]

    @functools.partial(
        pl.kernel,
        out_type=jax.ShapeDtypeStruct((NC, n, DW), jnp.float32),
        mesh=_mesh(),
        compiler_params=_SC_PARAMS,
        scratch_types=[
            pltpu.VMEM_SHARED((n + DUMMY, DW), jnp.float32),
            pltpu.VMEM((ch, K), jnp.int32),
            pltpu.VMEM((K, DW), jnp.float32),
        ],
    )
    def deg_k(dst_hbm, zero_hbm, ones_hbm, out_hbm, acc, didx, ones_v):
        cid = lax.axis_index("c")
        sid = lax.axis_index("s")
        wid = sid * NC + cid
        pltpu.sync_copy(dst_hbm.at[wid], didx)
        pltpu.sync_copy(ones_hbm, ones_v)
        _per_tile_rows(sid, n, lambda base, sz: pltpu.sync_copy(
            zero_hbm.at[pl.ds(base, sz), :], acc.at[pl.ds(base, sz), :]))
        plsc.subcore_barrier()

        @pl.loop(0, ch)
        def _(ci):
            pltpu.sync_copy(ones_v, acc.at[didx.at[ci]], add=True)

        plsc.subcore_barrier()
        _per_tile_rows(sid, n, lambda base, sz: pltpu.sync_copy(
            acc.at[pl.ds(base, sz), :], out_hbm.at[cid, pl.ds(base, sz), :]))

    return deg_k(dst3, zeros_dw, ones_dw)


def _agg_partials(src3, dst3, hs, zeros_nd, n, d):
    ch = src3.shape[1]
    cb = 16
    nb = ch // cb
    assert ch % cb == 0 and cb % 2 == 0

    @functools.partial(
        pl.kernel,
        out_type=jax.ShapeDtypeStruct((NC, n, d), jnp.float32),
        mesh=_mesh(),
        compiler_params=_SC_PARAMS,
        scratch_types=[
            pltpu.VMEM_SHARED((n + DUMMY, d), jnp.float32),
            pltpu.VMEM_SHARED((n, d), jnp.float32),
            pltpu.VMEM((cb, K), jnp.int32),
            pltpu.VMEM((cb, K), jnp.int32),
            pltpu.VMEM((K, d), jnp.float32),
            pltpu.VMEM((K, d), jnp.float32),
            pltpu.SemaphoreType.DMA,
            pltpu.SemaphoreType.DMA,
        ],
    )
    def agg_k(src_hbm, dst_hbm, hs_hbm, zero_hbm, out_hbm,
              acc, hs_sp, sidx, didx, rows0, rows1, sem0, sem1):
        cid = lax.axis_index("c")
        sid = lax.axis_index("s")
        wid = sid * NC + cid
        _per_tile_rows(sid, n, lambda base, sz: pltpu.sync_copy(
            zero_hbm.at[pl.ds(base, sz), :], acc.at[pl.ds(base, sz), :]))
        _per_tile_rows(sid, n, lambda base, sz: pltpu.sync_copy(
            hs_hbm.at[pl.ds(base, sz), :], hs_sp.at[pl.ds(base, sz), :]))

        plsc.subcore_barrier()

        @pl.loop(0, nb)
        def _(b):
            boff = pl.multiple_of(b * cb, 8)
            pltpu.sync_copy(src_hbm.at[wid, pl.ds(boff, cb)], sidx)
            pltpu.sync_copy(dst_hbm.at[wid, pl.ds(boff, cb)], didx)
            pltpu.async_copy(hs_sp.at[sidx.at[0]], rows0, sem0)

            @pl.loop(0, cb, step=2)
            def _(ci):
                pltpu.async_copy(hs_sp.at[sidx.at[ci + 1]], rows1, sem1)
                pltpu.make_async_copy(hs_sp.at[sidx.at[ci]], rows0, sem0).wait()
                pltpu.sync_copy(rows0, acc.at[didx.at[ci]], add=True)

                @pl.when(ci + 2 < cb)
                def _():
                    pltpu.async_copy(hs_sp.at[sidx.at[ci + 2]], rows0, sem0)

                pltpu.make_async_copy(hs_sp.at[sidx.at[ci + 1]], rows1, sem1).wait()
                pltpu.sync_copy(rows1, acc.at[didx.at[ci + 1]], add=True)

        plsc.subcore_barrier()
        _per_tile_rows(sid, n, lambda base, sz: pltpu.sync_copy(
            acc.at[pl.ds(base, sz), :], out_hbm.at[cid, pl.ds(base, sz), :]))

    return agg_k(src3, dst3, hs, zeros_nd)


def _dinv_col(deg_ref):
    deg = deg_ref[0, :, 0:1] + deg_ref[1, :, 0:1] + 1.0
    return lax.rsqrt(deg)


def _tc_first(deg_p, x, w1):
    n = x.shape[0]
    dh = w1.shape[1]

    def body(deg_ref, x_ref, w_ref, o_ref):
        dinv = _dinv_col(deg_ref)
        h = jnp.dot(x_ref[...], w_ref[...], preferred_element_type=jnp.float32)
        o_ref[...] = h * dinv

    return pl.pallas_call(
        body, out_shape=jax.ShapeDtypeStruct((n, dh), jnp.float32)
    )(deg_p, x, w1)


def _tc_mid(deg_p, p1, h1s, b1):
    n = p1.shape[1]
    dh = p1.shape[2]

    def body(deg_ref, p_ref, hs_ref, b_ref, o_ref):
        dinv = _dinv_col(deg_ref)
        s = p_ref[0] + p_ref[1] + hs_ref[...]
        h = jnp.maximum(s * dinv + b_ref[...], 0.0)
        o_ref[...] = h * dinv

    return pl.pallas_call(
        body, out_shape=jax.ShapeDtypeStruct((n, dh), jnp.float32)
    )(deg_p, p1, h1s, b1)


def _tc_last(deg_p, p2, h2s, w2, b2):
    n = p2.shape[1]
    do = w2.shape[1]

    def body(deg_ref, p_ref, hs_ref, w_ref, b_ref, o_ref):
        dinv = _dinv_col(deg_ref)
        agg = (p_ref[0] + p_ref[1] + hs_ref[...]) * dinv
        o_ref[...] = jnp.dot(
            agg, w_ref[...], preferred_element_type=jnp.float32) + b_ref[...]

    return pl.pallas_call(
        body, out_shape=jax.ShapeDtypeStruct((n, do), jnp.float32)
    )(deg_p, p2, h2s, w2, b2)


def kernel(x, edge_index, W1, b1, W2, b2):
    n = x.shape[0]
    dh = W1.shape[1]
    do = W2.shape[1]
    e = edge_index.shape[1]

    nw = NC * NS
    ch = -(-e // (nw * K))
    ch = -(-ch // 16) * 16
    pad = nw * ch * K - e
    src = jnp.concatenate([edge_index[0], jnp.zeros((pad,), edge_index.dtype)])
    pad_dst = n + (jnp.arange(pad, dtype=edge_index.dtype) % DUMMY)
    dst = jnp.concatenate([edge_index[1], pad_dst])
    src3 = src.reshape(nw, ch, K)
    dst3 = dst.reshape(nw, ch, K)

    zeros_dw = jnp.zeros((n, DW), jnp.float32)
    ones_dw = jnp.ones((K, DW), jnp.float32)
    zeros_h = jnp.zeros((n, dh), jnp.float32)

    deg_p = _deg_partials(dst3, zeros_dw, ones_dw, n)
    h1s = _tc_first(deg_p, x, W1)
    p1 = _agg_partials(src3, dst3, h1s, zeros_h, n, dh)
    h2s = _tc_mid(deg_p, p1, h1s, b1)
    p2 = _agg_partials(src3, dst3, h2s, zeros_h, n, dh)
    return _tc_last(deg_p, p2, h2s, W2, b2)
